# baseline (speedup 1.0000x reference)
import jax
import jax.numpy as jnp
from jax.experimental import pallas as pl

TOP_K = 8
THRESHOLD = 1e-4
INV_TEMPERATURE = 10.0


def _graph_block_kernel(x_ref, o_ref):
    x = jnp.transpose(x_ref[0], (1, 0, 2))
    norm2 = jnp.sum(x * x, axis=-1, keepdims=True)
    xn = x * jax.lax.rsqrt(jnp.maximum(norm2, 1e-24))
    adj = jax.lax.dot_general(
        xn, xn, (((2,), (2,)), ((0,), (0,))),
        preferred_element_type=jnp.float32,
    )

    e = jnp.exp((adj - 1.0) * INV_TEMPERATURE)
    s = jnp.sum(e, axis=-1, keepdims=True)
    rh = 0.5 / s

    row = jax.lax.broadcasted_iota(jnp.int32, adj.shape, 1)
    col = jax.lax.broadcasted_iota(jnp.int32, adj.shape, 2)
    work = jnp.where(row == col, -jnp.inf, adj)
    for _ in range(TOP_K - 2):
        mx = jnp.max(work, axis=-1, keepdims=True)
        work = jnp.where(work < mx, work, -jnp.inf)
    t8 = jnp.max(work, axis=-1, keepdims=True)

    t8_l = jnp.transpose(t8, (0, 2, 1))
    rh_l = jnp.transpose(rh, (0, 2, 1))

    half_th = 0.5 * THRESHOLD
    p = e * rh
    q = e * rh_l
    p = jnp.where((adj >= t8) & (p > half_th), p, 0.0)
    q = jnp.where((adj >= t8_l) & (q > half_th), q, 0.0)
    o_ref[0] = p + q


def kernel(features):
    B, N, T, D = features.shape
    Tt = 64
    return pl.pallas_call(
        _graph_block_kernel,
        grid=(B, T // Tt),
        in_specs=[pl.BlockSpec((1, N, Tt, D), lambda b, t: (b, 0, t, 0))],
        out_specs=pl.BlockSpec((1, Tt, N, N), lambda b, t: (b, t, 0, 0)),
        out_shape=jax.ShapeDtypeStruct((B, T, N, N), jnp.float32),
    )(features)


# speedup vs baseline: 1.0972x; 1.0972x over previous
import jax
import jax.numpy as jnp
from jax.experimental import pallas as pl

TOP_K = 8
THRESHOLD = 1e-4
INV_TEMPERATURE = 10.0
CHUNK = 64


def _graph_block_kernel(x_ref, o_ref):
    Tt = x_ref.shape[2]
    for c in range(Tt // CHUNK):
        x = jnp.transpose(
            x_ref[0, :, c * CHUNK:(c + 1) * CHUNK, :], (1, 0, 2)
        )
        norm2 = jnp.sum(x * x, axis=-1, keepdims=True)
        xn = x * jax.lax.rsqrt(jnp.maximum(norm2, 1e-24))
        adj = jax.lax.dot_general(
            xn, xn, (((2,), (2,)), ((0,), (0,))),
            preferred_element_type=jnp.float32,
        )

        e = jnp.exp((adj - 1.0) * INV_TEMPERATURE)
        s = jnp.sum(e, axis=-1, keepdims=True)
        r = 1.0 / s

        row = jax.lax.broadcasted_iota(jnp.int32, e.shape, 1)
        col = jax.lax.broadcasted_iota(jnp.int32, e.shape, 2)
        work = jnp.where(row == col, -1.0, e)
        for _ in range(TOP_K - 2):
            mx = jnp.max(work, axis=-1, keepdims=True)
            work = jnp.where(work < mx, work, -1.0)
        t8 = jnp.max(work, axis=-1, keepdims=True)

        keep = (e >= t8) & (e > THRESHOLD * s)
        a = jnp.where(keep, e, 0.0) * r
        o_ref[0, c * CHUNK:(c + 1) * CHUNK] = (
            (a + jnp.transpose(a, (0, 2, 1))) * 0.5
        )


def kernel(features):
    B, N, T, D = features.shape
    Tt = 64
    return pl.pallas_call(
        _graph_block_kernel,
        grid=(B, T // Tt),
        in_specs=[pl.BlockSpec((1, N, Tt, D), lambda b, t: (b, 0, t, 0))],
        out_specs=pl.BlockSpec((1, Tt, N, N), lambda b, t: (b, t, 0, 0)),
        out_shape=jax.ShapeDtypeStruct((B, T, N, N), jnp.float32),
    )(features)
